# baseline (device time: 9128 ns/iter reference)
import jax
import jax.numpy as jnp
from jax import lax
from jax.experimental import pallas as pl
from jax.experimental.pallas import tpu as pltpu

N_DEV = 16


def kernel(x):
    m, n = x.shape

    def body(x_ref, out_ref, comm_ref, total_ref, send_sems, recv_sems):
        my = lax.axis_index("i")

        barrier_sem = pltpu.get_barrier_semaphore()
        for j in range(N_DEV):
            @pl.when(j < my)
            def _(j=j):
                pl.semaphore_signal(
                    barrier_sem, inc=1,
                    device_id=(j,), device_id_type=pl.DeviceIdType.MESH,
                )
        total_ref[:, :] = jnp.sum(x_ref[:, :], axis=0, keepdims=True)
        for c in range(N_DEV - 1):
            @pl.when(my == c)
            def _(c=c):
                pl.semaphore_wait(barrier_sem, N_DEV - 1 - c)

        for j in range(N_DEV):
            @pl.when(my < j)
            def _(j=j):
                rdma = pltpu.make_async_remote_copy(
                    src_ref=total_ref,
                    dst_ref=comm_ref.at[pl.ds(my, 1)],
                    send_sem=send_sems.at[j],
                    recv_sem=recv_sems.at[my],
                    device_id=(j,),
                    device_id_type=pl.DeviceIdType.MESH,
                )
                rdma.start()

        row = lax.broadcasted_iota(jnp.int32, (m, m), 0)
        col = lax.broadcasted_iota(jnp.int32, (m, m), 1)
        tri = (row >= col).astype(jnp.float32)
        cs = jnp.dot(tri, x_ref[:, :], preferred_element_type=jnp.float32)

        for j in range(N_DEV):
            @pl.when(j < my)
            def _(j=j):
                rdma = pltpu.make_async_remote_copy(
                    src_ref=total_ref,
                    dst_ref=comm_ref.at[pl.ds(j, 1)],
                    send_sem=send_sems.at[j],
                    recv_sem=recv_sems.at[j],
                    device_id=(j,),
                    device_id_type=pl.DeviceIdType.MESH,
                )
                rdma.wait_recv()

        idx = lax.broadcasted_iota(jnp.int32, (N_DEV, n), 0)
        contrib = jnp.where(idx < my, comm_ref[:, :], 0.0)
        carry = jnp.sum(contrib, axis=0, keepdims=True)
        out_ref[:, :] = cs + carry

        for j in range(N_DEV):
            @pl.when(my < j)
            def _(j=j):
                rdma = pltpu.make_async_remote_copy(
                    src_ref=total_ref,
                    dst_ref=comm_ref.at[pl.ds(my, 1)],
                    send_sem=send_sems.at[j],
                    recv_sem=recv_sems.at[my],
                    device_id=(j,),
                    device_id_type=pl.DeviceIdType.MESH,
                )
                rdma.wait_send()

    return pl.pallas_call(
        body,
        out_shape=jax.ShapeDtypeStruct((m, n), jnp.float32),
        in_specs=[pl.BlockSpec(memory_space=pltpu.VMEM)],
        out_specs=pl.BlockSpec(memory_space=pltpu.VMEM),
        scratch_shapes=[
            pltpu.VMEM((N_DEV, n), jnp.float32),
            pltpu.VMEM((1, n), jnp.float32),
            pltpu.SemaphoreType.DMA((N_DEV,)),
            pltpu.SemaphoreType.DMA((N_DEV,)),
        ],
        compiler_params=pltpu.CompilerParams(collective_id=0),
    )(x)


# device time: 4517 ns/iter; 2.0208x vs baseline; 2.0208x over previous
import jax
import jax.numpy as jnp
from jax import lax
from jax.experimental import pallas as pl
from jax.experimental.pallas import tpu as pltpu

N_DEV = 16


def kernel(x):
    m, n = x.shape

    def body(x_hbm, out_ref, x_ref, comm_ref, total_ref, copy_sem,
             send_sems, recv_sems):
        my = lax.axis_index("i")

        copy_in = pltpu.make_async_copy(x_hbm, x_ref, copy_sem)
        copy_in.start()

        barrier_sem = pltpu.get_barrier_semaphore()
        for j in range(N_DEV):
            @pl.when(j < my)
            def _(j=j):
                pl.semaphore_signal(
                    barrier_sem, inc=1,
                    device_id=(j,), device_id_type=pl.DeviceIdType.MESH,
                )
        copy_in.wait()
        total_ref[:, :] = jnp.sum(x_ref[:, :], axis=0, keepdims=True)
        for c in range(N_DEV - 1):
            @pl.when(my == c)
            def _(c=c):
                pl.semaphore_wait(barrier_sem, N_DEV - 1 - c)

        for j in range(N_DEV):
            @pl.when(my < j)
            def _(j=j):
                rdma = pltpu.make_async_remote_copy(
                    src_ref=total_ref,
                    dst_ref=comm_ref.at[pl.ds(my, 1)],
                    send_sem=send_sems.at[j],
                    recv_sem=recv_sems.at[my],
                    device_id=(j,),
                    device_id_type=pl.DeviceIdType.MESH,
                )
                rdma.start()

        row = lax.broadcasted_iota(jnp.int32, (m, m), 0)
        col = lax.broadcasted_iota(jnp.int32, (m, m), 1)
        tri = (row >= col).astype(jnp.float32)
        cs = jnp.dot(tri, x_ref[:, :], preferred_element_type=jnp.float32)

        for j in range(N_DEV):
            @pl.when(j < my)
            def _(j=j):
                rdma = pltpu.make_async_remote_copy(
                    src_ref=total_ref,
                    dst_ref=comm_ref.at[pl.ds(j, 1)],
                    send_sem=send_sems.at[j],
                    recv_sem=recv_sems.at[j],
                    device_id=(j,),
                    device_id_type=pl.DeviceIdType.MESH,
                )
                rdma.wait_recv()

        idx = lax.broadcasted_iota(jnp.int32, (N_DEV, n), 0)
        contrib = jnp.where(idx < my, comm_ref[:, :], 0.0)
        carry = jnp.sum(contrib, axis=0, keepdims=True)
        out_ref[:, :] = cs + carry

        for j in range(N_DEV):
            @pl.when(my < j)
            def _(j=j):
                rdma = pltpu.make_async_remote_copy(
                    src_ref=total_ref,
                    dst_ref=comm_ref.at[pl.ds(my, 1)],
                    send_sem=send_sems.at[j],
                    recv_sem=recv_sems.at[my],
                    device_id=(j,),
                    device_id_type=pl.DeviceIdType.MESH,
                )
                rdma.wait_send()

    return pl.pallas_call(
        body,
        out_shape=jax.ShapeDtypeStruct((m, n), jnp.float32),
        in_specs=[pl.BlockSpec(memory_space=pl.ANY)],
        out_specs=pl.BlockSpec(memory_space=pltpu.VMEM),
        scratch_shapes=[
            pltpu.VMEM((m, n), jnp.float32),
            pltpu.VMEM((N_DEV, n), jnp.float32),
            pltpu.VMEM((1, n), jnp.float32),
            pltpu.SemaphoreType.DMA,
            pltpu.SemaphoreType.DMA((N_DEV,)),
            pltpu.SemaphoreType.DMA((N_DEV,)),
        ],
        compiler_params=pltpu.CompilerParams(collective_id=0),
    )(x)
